# baseline (speedup 1.0000x reference)
import functools

import jax
import jax.numpy as jnp
from jax.experimental import pallas as pl
from jax.experimental.pallas import tpu as pltpu

F32 = jnp.float32
NEG = -1e9
T = 2048
D = 1024
H = 16
HD = 64
HALF = HD // 2
NS = 4
DI = 2048
NSDI = NS * DI
MDIM = 64
E = 8
DFF = 2048
BOT = 256
KATT = 204
EPS = 1e-6
THETA = 10000.0
BLK = 256
NT = T // BLK
BSEARCH_ITERS = 20


def _rms(x, w):
    r = jax.lax.rsqrt(jnp.mean(x * x, axis=-1, keepdims=True) + EPS)
    return x * r * w


def _silu(x):
    return x * jax.nn.sigmoid(x)


def _attn_proj_body(x_ref, n1_ref, wq_ref, wk_ref, wv_ref, q_ref, k_ref, v_ref):
    xn = _rms(x_ref[...], n1_ref[...])
    q = xn @ wq_ref[...]
    k = xn @ wk_ref[...]
    v_ref[...] = xn @ wv_ref[...]
    t0 = pl.program_id(0) * BLK
    tpos = (jax.lax.broadcasted_iota(jnp.int32, (BLK, D // 2), 0)
            + t0).astype(F32)
    lane = jax.lax.broadcasted_iota(jnp.int32, (BLK, D // 2), 1)
    j = (lane % HALF).astype(F32)
    freq = jnp.exp(j * (-jnp.log(THETA) / HALF))
    ang = tpos * freq
    c = jnp.cos(ang)
    s = jnp.sin(ang)

    def rope(t):
        t1 = t[:, : D // 2]
        t2 = t[:, D // 2:]
        return jnp.concatenate([t1 * c - t2 * s, t1 * s + t2 * c], axis=1)

    q_ref[...] = rope(q)
    k_ref[...] = rope(k)


def _attn_proj(x2d, n1, wq_p, wk_p, wv):
    n1r = n1.reshape(1, D)
    return pl.pallas_call(
        _attn_proj_body,
        grid=(NT,),
        in_specs=[
            pl.BlockSpec((BLK, D), lambda t: (t, 0)),
            pl.BlockSpec((1, D), lambda t: (0, 0)),
            pl.BlockSpec((D, D), lambda t: (0, 0)),
            pl.BlockSpec((D, D), lambda t: (0, 0)),
            pl.BlockSpec((D, D), lambda t: (0, 0)),
        ],
        out_specs=[
            pl.BlockSpec((BLK, D), lambda t: (t, 0)),
            pl.BlockSpec((BLK, D), lambda t: (t, 0)),
            pl.BlockSpec((BLK, D), lambda t: (t, 0)),
        ],
        out_shape=[jax.ShapeDtypeStruct((T, D), F32)] * 3,
    )(x2d, n1r, wq_p, wk_p, wv)


def _sm_proj_body(x_ref, n1_ref, win_ref, wk_ref, wv_ref, wq_ref,
                  u_ref, kk_ref, vv_ref, qq_ref):
    xn = _rms(x_ref[...], n1_ref[...])
    u_ref[...] = _silu(xn @ win_ref[...])
    kk_ref[...] = _silu(xn @ wk_ref[...])
    vv_ref[...] = xn @ wv_ref[...]
    qq_ref[...] = _silu(xn @ wq_ref[...])


def _sm_proj(x2d, n1, win, mwk, mwv, mwq):
    n1r = n1.reshape(1, D)
    return pl.pallas_call(
        _sm_proj_body,
        grid=(NT,),
        in_specs=[
            pl.BlockSpec((BLK, D), lambda t: (t, 0)),
            pl.BlockSpec((1, D), lambda t: (0, 0)),
            pl.BlockSpec((D, DI), lambda t: (0, 0)),
            pl.BlockSpec((D, MDIM), lambda t: (0, 0)),
            pl.BlockSpec((D, MDIM), lambda t: (0, 0)),
            pl.BlockSpec((D, MDIM), lambda t: (0, 0)),
        ],
        out_specs=[
            pl.BlockSpec((BLK, DI), lambda t: (t, 0)),
            pl.BlockSpec((BLK, MDIM), lambda t: (t, 0)),
            pl.BlockSpec((BLK, MDIM), lambda t: (t, 0)),
            pl.BlockSpec((BLK, MDIM), lambda t: (t, 0)),
        ],
        out_shape=[
            jax.ShapeDtypeStruct((T, DI), F32),
            jax.ShapeDtypeStruct((T, MDIM), F32),
            jax.ShapeDtypeStruct((T, MDIM), F32),
            jax.ShapeDtypeStruct((T, MDIM), F32),
        ],
    )(x2d, n1r, win, mwk, mwv, mwq)


def _ssm_scan_body(u_ref, alog_ref, hs_ref, carry):
    i = pl.program_id(0)

    @pl.when(i == 0)
    def _():
        carry[...] = jnp.zeros_like(carry)

    dec = jax.nn.sigmoid(alog_ref[...])
    u = u_ref[...]
    ut = jnp.concatenate([u] * NS, axis=1)
    w = (1.0 - dec) * ut
    dshift = dec
    shift = 1
    while shift < BLK:
        w = w + dshift * jnp.concatenate(
            [jnp.zeros((shift, NSDI), F32), w[:-shift, :]], axis=0)
        dshift = dshift * dshift
        shift *= 2
    ldec = jnp.log(dec)
    trow = (jax.lax.broadcasted_iota(jnp.int32, (BLK, 1), 0) + 1).astype(F32)
    pows = jnp.exp(trow * ldec)
    h = w + pows * carry[...]
    hs_ref[...] = h
    carry[...] = h[BLK - 1:BLK, :]


def _ssm_scan(u, alog_flat):
    return pl.pallas_call(
        _ssm_scan_body,
        grid=(NT,),
        in_specs=[
            pl.BlockSpec((BLK, DI), lambda t: (t, 0)),
            pl.BlockSpec((1, NSDI), lambda t: (0, 0)),
        ],
        out_specs=pl.BlockSpec((BLK, NSDI), lambda t: (t, 0)),
        out_shape=jax.ShapeDtypeStruct((T, NSDI), F32),
        scratch_shapes=[pltpu.VMEM((1, NSDI), F32)],
        compiler_params=pltpu.CompilerParams(
            dimension_semantics=("arbitrary",)),
    )(u, alog_flat)


def _mm_body(a_ref, b_ref, o_ref):
    k = pl.program_id(1)

    @pl.when(k == 0)
    def _():
        o_ref[...] = jnp.zeros_like(o_ref)

    o_ref[...] += a_ref[...] @ b_ref[...]


def _matmul(a, b, blk_k):
    t, kdim = a.shape
    n = b.shape[1]
    return pl.pallas_call(
        _mm_body,
        grid=(t // BLK, kdim // blk_k),
        in_specs=[
            pl.BlockSpec((BLK, blk_k), lambda i, k: (i, k)),
            pl.BlockSpec((blk_k, n), lambda i, k: (k, 0)),
        ],
        out_specs=pl.BlockSpec((BLK, n), lambda i, k: (i, 0)),
        out_shape=jax.ShapeDtypeStruct((t, n), F32),
        compiler_params=pltpu.CompilerParams(
            dimension_semantics=("parallel", "arbitrary")),
    )(a, b)


def _attn_body(q_ref, k_ref, v_ref, o_ref, *, tq, w):
    q = q_ref[0]
    k = k_ref[0]
    sc = jax.lax.dot_general(
        q, k, (((1,), (1,)), ((), ())),
        preferred_element_type=F32) * (1.0 / 8.0)
    q0 = tq * BLK
    qpos = jax.lax.broadcasted_iota(jnp.int32, (BLK, w), 0) + q0
    kpos = jax.lax.broadcasted_iota(jnp.int32, (BLK, w), 1)
    valid = kpos <= qpos
    sc = jnp.where(valid, sc, NEG)
    nvalid = (jax.lax.broadcasted_iota(jnp.int32, (BLK, 1), 0) + q0 + 1)
    lo0 = jnp.min(jnp.where(valid, sc, 1e30), axis=1, keepdims=True)
    hi0 = jnp.max(sc, axis=1, keepdims=True)
    ones = jnp.ones((w, 1), F32)

    def body(_, lohi):
        lo, hi = lohi
        mid = 0.5 * (lo + hi)
        ind = jnp.where(sc >= mid, 1.0, 0.0)
        cnt = jax.lax.dot_general(
            ind, ones, (((1,), (0,)), ((), ())), preferred_element_type=F32)
        pred = cnt >= float(KATT)
        return jnp.where(pred, mid, lo), jnp.where(pred, hi, mid)

    lo, _ = jax.lax.fori_loop(0, BSEARCH_ITERS, body, (lo0, hi0))
    th = jnp.where(nvalid <= KATT, NEG, lo)
    scm = jnp.where(sc >= th, sc, NEG)
    m = jnp.max(scm, axis=1, keepdims=True)
    p = jnp.exp(scm - m)
    p = p / jnp.sum(p, axis=1, keepdims=True)
    o_ref[0] = jax.lax.dot_general(
        p, v_ref[0], (((1,), (0,)), ((), ())), preferred_element_type=F32)


def _attn(qh, kh, vh):
    outs = []
    for tq in range(NT):
        w = (tq + 1) * BLK
        outs.append(pl.pallas_call(
            functools.partial(_attn_body, tq=tq, w=w),
            grid=(H,),
            in_specs=[
                pl.BlockSpec((1, BLK, HD), lambda h, tq=tq: (h, tq, 0)),
                pl.BlockSpec((1, w, HD), lambda h: (h, 0, 0)),
                pl.BlockSpec((1, w, HD), lambda h: (h, 0, 0)),
            ],
            out_specs=pl.BlockSpec((1, BLK, HD), lambda h: (h, 0, 0)),
            out_shape=jax.ShapeDtypeStruct((H, BLK, HD), F32),
            compiler_params=pltpu.CompilerParams(
                dimension_semantics=("arbitrary",)),
        )(qh, kh, vh))
    return jnp.concatenate(outs, axis=1)


def _mem_body(dec_ref, kk_ref, vv_ref, qq_ref, r_ref, mstate):
    i = pl.program_id(0)

    @pl.when(i == 0)
    def _():
        mstate[...] = jnp.zeros_like(mstate)

    md = jax.nn.sigmoid(dec_ref[0, 0])
    lmd = jnp.log(md)
    ii = jax.lax.broadcasted_iota(jnp.int32, (BLK, BLK), 0).astype(F32)
    jj = jax.lax.broadcasted_iota(jnp.int32, (BLK, BLK), 1).astype(F32)
    dm = jnp.where(ii >= jj, jnp.exp((ii - jj) * lmd), 0.0)
    kk = kk_ref[...]
    vv = vv_ref[...]
    qq = qq_ref[...]
    s = jax.lax.dot_general(
        qq, kk, (((1,), (1,)), ((), ())), preferred_element_type=F32) * dm
    r = s @ vv
    trow = jax.lax.broadcasted_iota(jnp.int32, (BLK, 1), 0).astype(F32)
    rowpow = jnp.exp((trow + 1.0) * lmd)
    r = r + (qq * rowpow) @ mstate[...]
    tailpow = jnp.exp((float(BLK - 1) - trow) * lmd)
    mnew = jnp.exp(float(BLK) * lmd) * mstate[...] + jax.lax.dot_general(
        kk * tailpow, vv, (((0,), (0,)), ((), ())), preferred_element_type=F32)
    mstate[...] = mnew
    r_ref[...] = r


def _mem(dec11, kk, vv, qq):
    return pl.pallas_call(
        _mem_body,
        grid=(NT,),
        in_specs=[
            pl.BlockSpec((1, 1), lambda t: (0, 0)),
            pl.BlockSpec((BLK, MDIM), lambda t: (t, 0)),
            pl.BlockSpec((BLK, MDIM), lambda t: (t, 0)),
            pl.BlockSpec((BLK, MDIM), lambda t: (t, 0)),
        ],
        out_specs=pl.BlockSpec((BLK, MDIM), lambda t: (t, 0)),
        out_shape=jax.ShapeDtypeStruct((T, MDIM), F32),
        scratch_shapes=[pltpu.VMEM((MDIM, MDIM), F32)],
        compiler_params=pltpu.CompilerParams(
            dimension_semantics=("arbitrary",)),
    )(dec11, kk, vv, qq)


def _fusion_body(x_ref, ssm_ref, ao_ref, rs_ref, wo_ref, mwo_ref,
                 fw1_ref, fb1_ref, fw2_ref, fb2_ref, n2_ref, wr_ref,
                 x1_ref, xn2_ref, g_ref):
    attn_out = ao_ref[...] @ wo_ref[...]
    mem_out = rs_ref[...] @ mwo_ref[...]
    ssm_out = ssm_ref[...]
    w1 = fw1_ref[...]
    g1 = _silu(ssm_out @ w1[:D, :] + attn_out @ w1[D:2 * D, :]
               + mem_out @ w1[2 * D:, :] + fb1_ref[...])
    lg = g1 @ fw2_ref[...] + fb2_ref[...]
    lane = jax.lax.broadcasted_iota(jnp.int32, (BLK, 128), 1)
    lg = jnp.where(lane < 3, lg, -1e30)
    mg = jnp.max(lg, axis=1, keepdims=True)
    eg = jnp.exp(lg - mg)
    gate = eg / jnp.sum(eg, axis=1, keepdims=True)
    fused = (ssm_out * gate[:, 0:1] + attn_out * gate[:, 1:2]
             + mem_out * gate[:, 2:3])
    x1 = x_ref[...] + fused
    x1_ref[...] = x1
    xn2 = _rms(x1, n2_ref[...])
    xn2_ref[...] = xn2
    rl = xn2 @ wr_ref[...]
    rl = jnp.where(lane < E, rl, -1e30)
    lane_f = lane.astype(F32)
    m1 = jnp.max(rl, axis=1, keepdims=True)
    i1 = jnp.min(jnp.where(rl == m1, lane_f, 1e9), axis=1, keepdims=True)
    rl2 = jnp.where(lane_f == i1, -1e30, rl)
    m2 = jnp.max(rl2, axis=1, keepdims=True)
    i2 = jnp.min(jnp.where(rl2 == m2, lane_f, 1e9), axis=1, keepdims=True)
    p1 = jax.nn.sigmoid(m1 - m2)
    g_ref[...] = jnp.where(lane_f == i1, p1, 0.0) + jnp.where(
        lane_f == i2, 1.0 - p1, 0.0)


def _fusion(x2d, ssm_out, ao_flat, rs, wo, mwo, fw1, fb1, fw2p, fb2p, n2, wrp):
    return pl.pallas_call(
        _fusion_body,
        grid=(NT,),
        in_specs=[
            pl.BlockSpec((BLK, D), lambda t: (t, 0)),
            pl.BlockSpec((BLK, D), lambda t: (t, 0)),
            pl.BlockSpec((BLK, D), lambda t: (t, 0)),
            pl.BlockSpec((BLK, MDIM), lambda t: (t, 0)),
            pl.BlockSpec((D, D), lambda t: (0, 0)),
            pl.BlockSpec((MDIM, D), lambda t: (0, 0)),
            pl.BlockSpec((3 * D, BOT), lambda t: (0, 0)),
            pl.BlockSpec((1, BOT), lambda t: (0, 0)),
            pl.BlockSpec((BOT, 128), lambda t: (0, 0)),
            pl.BlockSpec((1, 128), lambda t: (0, 0)),
            pl.BlockSpec((1, D), lambda t: (0, 0)),
            pl.BlockSpec((D, 128), lambda t: (0, 0)),
        ],
        out_specs=[
            pl.BlockSpec((BLK, D), lambda t: (t, 0)),
            pl.BlockSpec((BLK, D), lambda t: (t, 0)),
            pl.BlockSpec((BLK, 128), lambda t: (t, 0)),
        ],
        out_shape=[
            jax.ShapeDtypeStruct((T, D), F32),
            jax.ShapeDtypeStruct((T, D), F32),
            jax.ShapeDtypeStruct((T, 128), F32),
        ],
    )(x2d, ssm_out, ao_flat, rs, wo, mwo, fw1, fb1, fw2p, fb2p, n2, wrp)


def _moe_body(xn2_ref, g_ref, x1_ref, w1_ref, b1_ref, w2_ref, b2_ref, o_ref):
    e = pl.program_id(1)
    lane = jax.lax.broadcasted_iota(jnp.int32, (BLK, 128), 1)
    ge = jnp.sum(jnp.where(lane == e, g_ref[...], 0.0), axis=1, keepdims=True)
    h = _silu(xn2_ref[...] @ w1_ref[0] + b1_ref[0])
    y = h @ w2_ref[0] + b2_ref[0]
    contrib = ge * y

    @pl.when(e == 0)
    def _():
        o_ref[...] = x1_ref[...] + contrib

    @pl.when(e > 0)
    def _():
        o_ref[...] += contrib


def _moe(xn2, gates, x1, w1, b1r, w2, b2r):
    return pl.pallas_call(
        _moe_body,
        grid=(NT, E),
        in_specs=[
            pl.BlockSpec((BLK, D), lambda t, e: (t, 0)),
            pl.BlockSpec((BLK, 128), lambda t, e: (t, 0)),
            pl.BlockSpec((BLK, D), lambda t, e: (t, 0)),
            pl.BlockSpec((1, D, DFF), lambda t, e: (e, 0, 0)),
            pl.BlockSpec((1, 1, DFF), lambda t, e: (e, 0, 0)),
            pl.BlockSpec((1, DFF, D), lambda t, e: (e, 0, 0)),
            pl.BlockSpec((1, 1, D), lambda t, e: (e, 0, 0)),
        ],
        out_specs=pl.BlockSpec((BLK, D), lambda t, e: (t, 0)),
        out_shape=jax.ShapeDtypeStruct((T, D), F32),
        compiler_params=pltpu.CompilerParams(
            dimension_semantics=("parallel", "arbitrary")),
    )(xn2, gates, x1, w1, b1r, w2, b2r)


def _permute_halves(w):
    r = w.reshape(D, H, HD)
    return jnp.concatenate(
        [r[:, :, :HALF].reshape(D, D // 2), r[:, :, HALF:].reshape(D, D // 2)],
        axis=1)


def _to_heads_halves(qp):
    a = qp[:, : D // 2].reshape(T, H, HALF)
    b = qp[:, D // 2:].reshape(T, H, HALF)
    return jnp.concatenate([a, b], axis=-1).transpose(1, 0, 2)


def kernel(x, norm1_w, norm2_w, ssm_Win, ssm_alog, ssm_Wout,
           attn_Wq, attn_Wk, attn_Wv, attn_Wo,
           mem_Wk, mem_Wv, mem_Wq, mem_Wo, mem_decay,
           fus_W1, fus_b1, fus_W2, fus_b2,
           moe_Wr, moe_W1, moe_b1, moe_W2, moe_b2):
    x2d = x.reshape(T, D)

    qp, kp, vflat = _attn_proj(
        x2d, norm1_w, _permute_halves(attn_Wq), _permute_halves(attn_Wk),
        attn_Wv)
    u, kk, vv, qq = _sm_proj(x2d, norm1_w, ssm_Win, mem_Wk, mem_Wv, mem_Wq)

    hs = _ssm_scan(u, ssm_alog.reshape(1, NSDI))
    ssm_out = _matmul(hs, ssm_Wout, 1024)

    qh = _to_heads_halves(qp)
    kh = _to_heads_halves(kp)
    vh = vflat.reshape(T, H, HD).transpose(1, 0, 2)
    ao = _attn(qh, kh, vh)
    ao_flat = ao.transpose(1, 0, 2).reshape(T, D)

    rs = _mem(mem_decay.reshape(1, 1), kk, vv, qq)

    fw2p = jnp.pad(fus_W2, ((0, 0), (0, 128 - 3)))
    fb2p = jnp.pad(fus_b2, (0, 128 - 3)).reshape(1, 128)
    wrp = jnp.pad(moe_Wr, ((0, 0), (0, 128 - E)))
    x1, xn2, gates = _fusion(
        x2d, ssm_out, ao_flat, rs, attn_Wo, mem_Wo, fus_W1,
        fus_b1.reshape(1, BOT), fw2p, fb2p, norm2_w.reshape(1, D), wrp)

    out = _moe(xn2, gates, x1, moe_W1, moe_b1.reshape(E, 1, DFF),
               moe_W2, moe_b2.reshape(E, 1, D))
    return out.reshape(1, T, D)


# speedup vs baseline: 1.2687x; 1.2687x over previous
import functools

import jax
import jax.numpy as jnp
from jax.experimental import pallas as pl
from jax.experimental.pallas import tpu as pltpu

F32 = jnp.float32
NEG = -1e9
T = 2048
D = 1024
H = 16
HD = 64
HALF = HD // 2
NS = 4
DI = 2048
NSDI = NS * DI
MDIM = 64
E = 8
DFF = 2048
BOT = 256
KATT = 204
EPS = 1e-6
THETA = 10000.0
BLK = 256
NT = T // BLK
BSEARCH_ITERS = 20


def _rms(x, w):
    r = jax.lax.rsqrt(jnp.mean(x * x, axis=-1, keepdims=True) + EPS)
    return x * r * w


def _silu(x):
    return x * jax.nn.sigmoid(x)


def _attn_proj_body(x_ref, n1_ref, wq_ref, wk_ref, wv_ref, q_ref, k_ref, v_ref):
    xn = _rms(x_ref[...], n1_ref[...])
    q = xn @ wq_ref[...]
    k = xn @ wk_ref[...]
    v_ref[...] = xn @ wv_ref[...]
    t0 = pl.program_id(0) * BLK
    tpos = (jax.lax.broadcasted_iota(jnp.int32, (BLK, D // 2), 0)
            + t0).astype(F32)
    lane = jax.lax.broadcasted_iota(jnp.int32, (BLK, D // 2), 1)
    j = (lane % HALF).astype(F32)
    freq = jnp.exp(j * (-jnp.log(THETA) / HALF))
    ang = tpos * freq
    c = jnp.cos(ang)
    s = jnp.sin(ang)

    def rope(t):
        t1 = t[:, : D // 2]
        t2 = t[:, D // 2:]
        return jnp.concatenate([t1 * c - t2 * s, t1 * s + t2 * c], axis=1)

    q_ref[...] = rope(q)
    k_ref[...] = rope(k)


def _attn_proj(x2d, n1, wq_p, wk_p, wv):
    n1r = n1.reshape(1, D)
    return pl.pallas_call(
        _attn_proj_body,
        grid=(NT,),
        in_specs=[
            pl.BlockSpec((BLK, D), lambda t: (t, 0)),
            pl.BlockSpec((1, D), lambda t: (0, 0)),
            pl.BlockSpec((D, D), lambda t: (0, 0)),
            pl.BlockSpec((D, D), lambda t: (0, 0)),
            pl.BlockSpec((D, D), lambda t: (0, 0)),
        ],
        out_specs=[
            pl.BlockSpec((BLK, D), lambda t: (t, 0)),
            pl.BlockSpec((BLK, D), lambda t: (t, 0)),
            pl.BlockSpec((BLK, D), lambda t: (t, 0)),
        ],
        out_shape=[jax.ShapeDtypeStruct((T, D), F32)] * 3,
    )(x2d, n1r, wq_p, wk_p, wv)


def _sm_proj_body(x_ref, n1_ref, win_ref, wk_ref, wv_ref, wq_ref,
                  u_ref, kk_ref, vv_ref, qq_ref):
    xn = _rms(x_ref[...], n1_ref[...])
    u_ref[...] = _silu(xn @ win_ref[...])
    kk_ref[...] = _silu(xn @ wk_ref[...])
    vv_ref[...] = xn @ wv_ref[...]
    qq_ref[...] = _silu(xn @ wq_ref[...])


def _sm_proj(x2d, n1, win, mwk, mwv, mwq):
    n1r = n1.reshape(1, D)
    return pl.pallas_call(
        _sm_proj_body,
        grid=(NT,),
        in_specs=[
            pl.BlockSpec((BLK, D), lambda t: (t, 0)),
            pl.BlockSpec((1, D), lambda t: (0, 0)),
            pl.BlockSpec((D, DI), lambda t: (0, 0)),
            pl.BlockSpec((D, MDIM), lambda t: (0, 0)),
            pl.BlockSpec((D, MDIM), lambda t: (0, 0)),
            pl.BlockSpec((D, MDIM), lambda t: (0, 0)),
        ],
        out_specs=[
            pl.BlockSpec((BLK, DI), lambda t: (t, 0)),
            pl.BlockSpec((BLK, MDIM), lambda t: (t, 0)),
            pl.BlockSpec((BLK, MDIM), lambda t: (t, 0)),
            pl.BlockSpec((BLK, MDIM), lambda t: (t, 0)),
        ],
        out_shape=[
            jax.ShapeDtypeStruct((T, DI), F32),
            jax.ShapeDtypeStruct((T, MDIM), F32),
            jax.ShapeDtypeStruct((T, MDIM), F32),
            jax.ShapeDtypeStruct((T, MDIM), F32),
        ],
    )(x2d, n1r, win, mwk, mwv, mwq)


def _ssm_scan_body(u_ref, alog_ref, hs_ref, carry):
    i = pl.program_id(0)

    @pl.when(i == 0)
    def _():
        carry[...] = jnp.zeros_like(carry)

    dec = jax.nn.sigmoid(alog_ref[...])
    u = u_ref[...]
    ut = jnp.concatenate([u] * NS, axis=1)
    w = (1.0 - dec) * ut
    dshift = dec
    shift = 1
    while shift < BLK:
        w = w + dshift * jnp.concatenate(
            [jnp.zeros((shift, NSDI), F32), w[:-shift, :]], axis=0)
        dshift = dshift * dshift
        shift *= 2
    ldec = jnp.log(dec)
    trow = (jax.lax.broadcasted_iota(jnp.int32, (BLK, 1), 0) + 1).astype(F32)
    pows = jnp.exp(trow * ldec)
    h = w + pows * carry[...]
    hs_ref[...] = h
    carry[...] = h[BLK - 1:BLK, :]


def _ssm_scan(u, alog_flat):
    return pl.pallas_call(
        _ssm_scan_body,
        grid=(NT,),
        in_specs=[
            pl.BlockSpec((BLK, DI), lambda t: (t, 0)),
            pl.BlockSpec((1, NSDI), lambda t: (0, 0)),
        ],
        out_specs=pl.BlockSpec((BLK, NSDI), lambda t: (t, 0)),
        out_shape=jax.ShapeDtypeStruct((T, NSDI), F32),
        scratch_shapes=[pltpu.VMEM((1, NSDI), F32)],
        compiler_params=pltpu.CompilerParams(
            dimension_semantics=("arbitrary",)),
    )(u, alog_flat)


def _mm_body(a_ref, b_ref, o_ref):
    k = pl.program_id(1)

    @pl.when(k == 0)
    def _():
        o_ref[...] = jnp.zeros_like(o_ref)

    o_ref[...] += a_ref[...] @ b_ref[...]


def _matmul(a, b, blk_k):
    t, kdim = a.shape
    n = b.shape[1]
    return pl.pallas_call(
        _mm_body,
        grid=(t // BLK, kdim // blk_k),
        in_specs=[
            pl.BlockSpec((BLK, blk_k), lambda i, k: (i, k)),
            pl.BlockSpec((blk_k, n), lambda i, k: (k, 0)),
        ],
        out_specs=pl.BlockSpec((BLK, n), lambda i, k: (i, 0)),
        out_shape=jax.ShapeDtypeStruct((t, n), F32),
        compiler_params=pltpu.CompilerParams(
            dimension_semantics=("parallel", "arbitrary")),
    )(a, b)


def _attn_body(q_ref, k_ref, v_ref, o_ref, *, tq, w):
    q = q_ref[0]
    k = k_ref[0]
    sc = jax.lax.dot_general(
        q, k, (((1,), (1,)), ((), ())),
        preferred_element_type=F32) * (1.0 / 8.0)
    q0 = tq * BLK
    qpos = jax.lax.broadcasted_iota(jnp.int32, (BLK, w), 0) + q0
    kpos = jax.lax.broadcasted_iota(jnp.int32, (BLK, w), 1)
    valid = kpos <= qpos
    sc = jnp.where(valid, sc, NEG)
    nvalid = (jax.lax.broadcasted_iota(jnp.int32, (BLK, 1), 0) + q0 + 1)
    lo0 = jnp.min(jnp.where(valid, sc, 1e30), axis=1, keepdims=True)
    hi0 = jnp.max(sc, axis=1, keepdims=True)
    def body(_, lohi):
        lo, hi = lohi
        mid = 0.5 * (lo + hi)
        cnt = jnp.sum((sc >= mid).astype(F32), axis=1, keepdims=True)
        pred = cnt >= float(KATT)
        return jnp.where(pred, mid, lo), jnp.where(pred, hi, mid)

    lo, _ = jax.lax.fori_loop(0, BSEARCH_ITERS, body, (lo0, hi0))
    th = jnp.where(nvalid <= KATT, NEG, lo)
    scm = jnp.where(sc >= th, sc, NEG)
    m = jnp.max(scm, axis=1, keepdims=True)
    p = jnp.exp(scm - m)
    p = p / jnp.sum(p, axis=1, keepdims=True)
    o_ref[0] = jax.lax.dot_general(
        p, v_ref[0], (((1,), (0,)), ((), ())), preferred_element_type=F32)


def _attn(qh, kh, vh):
    outs = []
    for tq in range(NT):
        w = (tq + 1) * BLK
        outs.append(pl.pallas_call(
            functools.partial(_attn_body, tq=tq, w=w),
            grid=(H,),
            in_specs=[
                pl.BlockSpec((1, BLK, HD), lambda h, tq=tq: (h, tq, 0)),
                pl.BlockSpec((1, w, HD), lambda h: (h, 0, 0)),
                pl.BlockSpec((1, w, HD), lambda h: (h, 0, 0)),
            ],
            out_specs=pl.BlockSpec((1, BLK, HD), lambda h: (h, 0, 0)),
            out_shape=jax.ShapeDtypeStruct((H, BLK, HD), F32),
            compiler_params=pltpu.CompilerParams(
                dimension_semantics=("arbitrary",)),
        )(qh, kh, vh))
    return jnp.concatenate(outs, axis=1)


def _mem_body(dec_ref, kk_ref, vv_ref, qq_ref, r_ref, mstate):
    i = pl.program_id(0)

    @pl.when(i == 0)
    def _():
        mstate[...] = jnp.zeros_like(mstate)

    md = jax.nn.sigmoid(dec_ref[0, 0])
    lmd = jnp.log(md)
    ii = jax.lax.broadcasted_iota(jnp.int32, (BLK, BLK), 0).astype(F32)
    jj = jax.lax.broadcasted_iota(jnp.int32, (BLK, BLK), 1).astype(F32)
    dm = jnp.where(ii >= jj, jnp.exp((ii - jj) * lmd), 0.0)
    kk = kk_ref[...]
    vv = vv_ref[...]
    qq = qq_ref[...]
    s = jax.lax.dot_general(
        qq, kk, (((1,), (1,)), ((), ())), preferred_element_type=F32) * dm
    r = s @ vv
    trow = jax.lax.broadcasted_iota(jnp.int32, (BLK, 1), 0).astype(F32)
    rowpow = jnp.exp((trow + 1.0) * lmd)
    r = r + (qq * rowpow) @ mstate[...]
    tailpow = jnp.exp((float(BLK - 1) - trow) * lmd)
    mnew = jnp.exp(float(BLK) * lmd) * mstate[...] + jax.lax.dot_general(
        kk * tailpow, vv, (((0,), (0,)), ((), ())), preferred_element_type=F32)
    mstate[...] = mnew
    r_ref[...] = r


def _mem(dec11, kk, vv, qq):
    return pl.pallas_call(
        _mem_body,
        grid=(NT,),
        in_specs=[
            pl.BlockSpec((1, 1), lambda t: (0, 0)),
            pl.BlockSpec((BLK, MDIM), lambda t: (t, 0)),
            pl.BlockSpec((BLK, MDIM), lambda t: (t, 0)),
            pl.BlockSpec((BLK, MDIM), lambda t: (t, 0)),
        ],
        out_specs=pl.BlockSpec((BLK, MDIM), lambda t: (t, 0)),
        out_shape=jax.ShapeDtypeStruct((T, MDIM), F32),
        scratch_shapes=[pltpu.VMEM((MDIM, MDIM), F32)],
        compiler_params=pltpu.CompilerParams(
            dimension_semantics=("arbitrary",)),
    )(dec11, kk, vv, qq)


def _fusion_body(x_ref, ssm_ref, ao_ref, rs_ref, wo_ref, mwo_ref,
                 fw1_ref, fb1_ref, fw2_ref, fb2_ref, n2_ref, wr_ref,
                 x1_ref, xn2_ref, g_ref):
    attn_out = ao_ref[...] @ wo_ref[...]
    mem_out = rs_ref[...] @ mwo_ref[...]
    ssm_out = ssm_ref[...]
    w1 = fw1_ref[...]
    g1 = _silu(ssm_out @ w1[:D, :] + attn_out @ w1[D:2 * D, :]
               + mem_out @ w1[2 * D:, :] + fb1_ref[...])
    lg = g1 @ fw2_ref[...] + fb2_ref[...]
    lane = jax.lax.broadcasted_iota(jnp.int32, (BLK, 128), 1)
    lg = jnp.where(lane < 3, lg, -1e30)
    mg = jnp.max(lg, axis=1, keepdims=True)
    eg = jnp.exp(lg - mg)
    gate = eg / jnp.sum(eg, axis=1, keepdims=True)
    fused = (ssm_out * gate[:, 0:1] + attn_out * gate[:, 1:2]
             + mem_out * gate[:, 2:3])
    x1 = x_ref[...] + fused
    x1_ref[...] = x1
    xn2 = _rms(x1, n2_ref[...])
    xn2_ref[...] = xn2
    rl = xn2 @ wr_ref[...]
    rl = jnp.where(lane < E, rl, -1e30)
    lane_f = lane.astype(F32)
    m1 = jnp.max(rl, axis=1, keepdims=True)
    i1 = jnp.min(jnp.where(rl == m1, lane_f, 1e9), axis=1, keepdims=True)
    rl2 = jnp.where(lane_f == i1, -1e30, rl)
    m2 = jnp.max(rl2, axis=1, keepdims=True)
    i2 = jnp.min(jnp.where(rl2 == m2, lane_f, 1e9), axis=1, keepdims=True)
    p1 = jax.nn.sigmoid(m1 - m2)
    g_ref[...] = jnp.where(lane_f == i1, p1, 0.0) + jnp.where(
        lane_f == i2, 1.0 - p1, 0.0)


def _fusion(x2d, ssm_out, ao_flat, rs, wo, mwo, fw1, fb1, fw2p, fb2p, n2, wrp):
    return pl.pallas_call(
        _fusion_body,
        grid=(NT,),
        in_specs=[
            pl.BlockSpec((BLK, D), lambda t: (t, 0)),
            pl.BlockSpec((BLK, D), lambda t: (t, 0)),
            pl.BlockSpec((BLK, D), lambda t: (t, 0)),
            pl.BlockSpec((BLK, MDIM), lambda t: (t, 0)),
            pl.BlockSpec((D, D), lambda t: (0, 0)),
            pl.BlockSpec((MDIM, D), lambda t: (0, 0)),
            pl.BlockSpec((3 * D, BOT), lambda t: (0, 0)),
            pl.BlockSpec((1, BOT), lambda t: (0, 0)),
            pl.BlockSpec((BOT, 128), lambda t: (0, 0)),
            pl.BlockSpec((1, 128), lambda t: (0, 0)),
            pl.BlockSpec((1, D), lambda t: (0, 0)),
            pl.BlockSpec((D, 128), lambda t: (0, 0)),
        ],
        out_specs=[
            pl.BlockSpec((BLK, D), lambda t: (t, 0)),
            pl.BlockSpec((BLK, D), lambda t: (t, 0)),
            pl.BlockSpec((BLK, 128), lambda t: (t, 0)),
        ],
        out_shape=[
            jax.ShapeDtypeStruct((T, D), F32),
            jax.ShapeDtypeStruct((T, D), F32),
            jax.ShapeDtypeStruct((T, 128), F32),
        ],
    )(x2d, ssm_out, ao_flat, rs, wo, mwo, fw1, fb1, fw2p, fb2p, n2, wrp)


def _moe_body(xn2_ref, g_ref, x1_ref, w1_ref, b1_ref, w2_ref, b2_ref, o_ref):
    e = pl.program_id(1)
    lane = jax.lax.broadcasted_iota(jnp.int32, (BLK, 128), 1)
    ge = jnp.sum(jnp.where(lane == e, g_ref[...], 0.0), axis=1, keepdims=True)
    h = _silu(xn2_ref[...] @ w1_ref[0] + b1_ref[0])
    y = h @ w2_ref[0] + b2_ref[0]
    contrib = ge * y

    @pl.when(e == 0)
    def _():
        o_ref[...] = x1_ref[...] + contrib

    @pl.when(e > 0)
    def _():
        o_ref[...] += contrib


def _moe(xn2, gates, x1, w1, b1r, w2, b2r):
    return pl.pallas_call(
        _moe_body,
        grid=(NT, E),
        in_specs=[
            pl.BlockSpec((BLK, D), lambda t, e: (t, 0)),
            pl.BlockSpec((BLK, 128), lambda t, e: (t, 0)),
            pl.BlockSpec((BLK, D), lambda t, e: (t, 0)),
            pl.BlockSpec((1, D, DFF), lambda t, e: (e, 0, 0)),
            pl.BlockSpec((1, 1, DFF), lambda t, e: (e, 0, 0)),
            pl.BlockSpec((1, DFF, D), lambda t, e: (e, 0, 0)),
            pl.BlockSpec((1, 1, D), lambda t, e: (e, 0, 0)),
        ],
        out_specs=pl.BlockSpec((BLK, D), lambda t, e: (t, 0)),
        out_shape=jax.ShapeDtypeStruct((T, D), F32),
        compiler_params=pltpu.CompilerParams(
            dimension_semantics=("parallel", "arbitrary")),
    )(xn2, gates, x1, w1, b1r, w2, b2r)


def _permute_halves(w):
    r = w.reshape(D, H, HD)
    return jnp.concatenate(
        [r[:, :, :HALF].reshape(D, D // 2), r[:, :, HALF:].reshape(D, D // 2)],
        axis=1)


def _to_heads_halves(qp):
    a = qp[:, : D // 2].reshape(T, H, HALF)
    b = qp[:, D // 2:].reshape(T, H, HALF)
    return jnp.concatenate([a, b], axis=-1).transpose(1, 0, 2)


def kernel(x, norm1_w, norm2_w, ssm_Win, ssm_alog, ssm_Wout,
           attn_Wq, attn_Wk, attn_Wv, attn_Wo,
           mem_Wk, mem_Wv, mem_Wq, mem_Wo, mem_decay,
           fus_W1, fus_b1, fus_W2, fus_b2,
           moe_Wr, moe_W1, moe_b1, moe_W2, moe_b2):
    x2d = x.reshape(T, D)

    qp, kp, vflat = _attn_proj(
        x2d, norm1_w, _permute_halves(attn_Wq), _permute_halves(attn_Wk),
        attn_Wv)
    u, kk, vv, qq = _sm_proj(x2d, norm1_w, ssm_Win, mem_Wk, mem_Wv, mem_Wq)

    hs = _ssm_scan(u, ssm_alog.reshape(1, NSDI))
    ssm_out = _matmul(hs, ssm_Wout, 1024)

    qh = _to_heads_halves(qp)
    kh = _to_heads_halves(kp)
    vh = vflat.reshape(T, H, HD).transpose(1, 0, 2)
    ao = _attn(qh, kh, vh)
    ao_flat = ao.transpose(1, 0, 2).reshape(T, D)

    rs = _mem(mem_decay.reshape(1, 1), kk, vv, qq)

    fw2p = jnp.pad(fus_W2, ((0, 0), (0, 128 - 3)))
    fb2p = jnp.pad(fus_b2, (0, 128 - 3)).reshape(1, 128)
    wrp = jnp.pad(moe_Wr, ((0, 0), (0, 128 - E)))
    x1, xn2, gates = _fusion(
        x2d, ssm_out, ao_flat, rs, attn_Wo, mem_Wo, fus_W1,
        fus_b1.reshape(1, BOT), fw2p, fb2p, norm2_w.reshape(1, D), wrp)

    out = _moe(xn2, gates, x1, moe_W1, moe_b1.reshape(E, 1, DFF),
               moe_W2, moe_b2.reshape(E, 1, D))
    return out.reshape(1, T, D)
